# Initial kernel scaffold; baseline (speedup 1.0000x reference)
import jax
import jax.numpy as jnp
from jax.experimental import pallas as pl


def kernel(input_ids, weight, token_type_embeddings, position_embeddings, gamma, beta):
    raise NotImplementedError("write your pallas kernel here")


# speedup vs baseline: 9.2606x; 9.2606x over previous
import functools

import jax
import jax.numpy as jnp
from jax import lax
from jax.experimental import pallas as pl
from jax.experimental.pallas import tpu as pltpu
from jax.experimental.pallas import tpu_sc as plsc

_EPS = 1e-12
_NC = 2
_NS = 16
_NW = _NC * _NS


def _sc_gather(ids, table, chunk=320):
    n, = ids.shape
    _, d = table.shape
    per_w = n // _NW
    assert n % _NW == 0 and per_w % (2 * chunk) == 0
    nch = per_w // chunk
    npairs = nch // 2
    mesh = plsc.VectorSubcoreMesh(core_axis_name="c", subcore_axis_name="s")

    @functools.partial(
        pl.kernel,
        mesh=mesh,
        out_type=jax.ShapeDtypeStruct((n, d), jnp.float32),
        scratch_types=[
            pltpu.VMEM((per_w,), jnp.int32),
            pltpu.VMEM((chunk, d), jnp.float32),
            pltpu.VMEM((chunk, d), jnp.float32),
            pltpu.SemaphoreType.DMA,
            pltpu.SemaphoreType.DMA,
            pltpu.SemaphoreType.DMA,
            pltpu.SemaphoreType.DMA,
        ],
    )
    def k(idx_hbm, table_hbm, out_hbm, idx_v, rows0, rows1, sg0, sg1, ss0, ss1):
        wid = lax.axis_index("s") * _NC + lax.axis_index("c")
        base = wid * per_w
        pltpu.sync_copy(idx_hbm.at[pl.ds(base, per_w)], idx_v)

        def g_desc(c, rows, sem):
            return pltpu.make_async_copy(
                table_hbm.at[idx_v.at[pl.ds(c * chunk, chunk)]], rows, sem)

        def s_desc(c, rows, sem):
            return pltpu.make_async_copy(
                rows, out_hbm.at[pl.ds(base + c * chunk, chunk)], sem)

        g_desc(0, rows0, sg0).start()

        def pair(p, carry):
            c0 = 2 * p
            c1 = c0 + 1
            g_desc(c0, rows0, sg0).wait()

            @pl.when(p > 0)
            def _():
                s_desc(c0 - 1, rows1, ss1).wait()

            g_desc(c1, rows1, sg1).start()
            s_desc(c0, rows0, ss0).start()
            g_desc(c1, rows1, sg1).wait()
            s_desc(c0, rows0, ss0).wait()

            @pl.when(p + 1 < npairs)
            def _():
                g_desc(c0 + 2, rows0, sg0).start()

            s_desc(c1, rows1, ss1).start()
            return carry

        lax.fori_loop(0, npairs, pair, 0)
        s_desc(nch - 1, rows1, ss1).wait()

    return k(ids, table)


def _tc_bias_layernorm(x, pos, tt0, gamma, beta, bb=16):
    b, l, d = x.shape

    def body(x_ref, pos_ref, tt_ref, g_ref, b_ref, o_ref):
        xb = x_ref[...] + pos_ref[...] + tt_ref[...]
        mean = jnp.mean(xb, axis=-1, keepdims=True)
        xc = xb - mean
        var = jnp.mean(xc * xc, axis=-1, keepdims=True)
        o_ref[...] = xc * lax.rsqrt(var + _EPS) * g_ref[...] + b_ref[...]

    return pl.pallas_call(
        body,
        grid=(b // bb,),
        in_specs=[
            pl.BlockSpec((bb, l, d), lambda i: (i, 0, 0)),
            pl.BlockSpec((1, l, d), lambda i: (0, 0, 0)),
            pl.BlockSpec((1, 1, d), lambda i: (0, 0, 0)),
            pl.BlockSpec((1, 1, d), lambda i: (0, 0, 0)),
            pl.BlockSpec((1, 1, d), lambda i: (0, 0, 0)),
        ],
        out_specs=pl.BlockSpec((bb, l, d), lambda i: (i, 0, 0)),
        out_shape=jax.ShapeDtypeStruct((b, l, d), jnp.float32),
    )(x, pos, tt0, gamma, beta)


def kernel(input_ids, weight, token_type_embeddings, position_embeddings, gamma, beta):
    b, l = input_ids.shape
    _, d = weight.shape
    ids = input_ids.reshape(-1).astype(jnp.int32)
    gathered = _sc_gather(ids, weight)
    x = gathered.reshape(b, l, d)
    pos = position_embeddings[:l].reshape(1, l, d)
    tt0 = token_type_embeddings[0].reshape(1, 1, d)
    return _tc_bias_layernorm(x, pos, tt0,
                              gamma.reshape(1, 1, d), beta.reshape(1, 1, d))
